# baseline (speedup 1.0000x reference)
import functools

import jax
import jax.numpy as jnp
import numpy as np
from jax import lax
from jax.experimental import pallas as pl
from jax.experimental.pallas import tpu as pltpu
from jax.experimental.pallas import tpu_sc as plsc

B = 16384
F = 26
D = 16
NW = 32
ROWS_W = B // NW
CHUNK = 64
NCHUNK = ROWS_W // CHUNK
G = 128
NG = CHUNK * F // G
IDX_W = ROWS_W * F

_mesh = plsc.VectorSubcoreMesh(core_axis_name="c", subcore_axis_name="s")


@functools.partial(
    pl.kernel,
    mesh=_mesh,
    out_type=jax.ShapeDtypeStruct((B,), jnp.float32),
    compiler_params=pltpu.CompilerParams(use_tc_tiling_on_sc=False, needs_layout_passes=False),
    scratch_types=[
        pltpu.VMEM((IDX_W // G, G), jnp.int32),
        pltpu.VMEM((CHUNK * F, D), jnp.float32),
        pltpu.VMEM((CHUNK * F + 16,), jnp.float32),
        pltpu.VMEM((ROWS_W + 16,), jnp.float32),
        pltpu.VMEM((16,), jnp.float32),
        pltpu.VMEM((16,), jnp.int32),
        pltpu.VMEM((16,), jnp.float32),
        pltpu.SemaphoreType.DMA,
        pltpu.SemaphoreType.DMA,
    ],
)
def _fm_sc(idx_hbm, t1_hbm, t2_hbm, bias_hbm, lane_hbm, tmask_hbm, out_hbm,
           idx_v, rows_v, lin_v, out_v, bias_v, lane_v, tmask_v, sem2, sem1):
    wid = lax.axis_index("s") * 2 + lax.axis_index("c")
    pltpu.sync_copy(idx_hbm.at[wid], idx_v)
    pltpu.sync_copy(bias_hbm, bias_v)
    pltpu.sync_copy(lane_hbm, lane_v)
    pltpu.sync_copy(tmask_hbm, tmask_v)
    lin_v[pl.ds(CHUNK * F, 16)] = jnp.zeros((16,), jnp.float32)

    def chunk_body(c, carry0):
        cps = []
        for g in range(NG):
            gg = c * NG + g
            cps.append(pltpu.async_copy(
                t2_hbm.at[idx_v.at[gg]], rows_v.at[pl.ds(g * G, G)], sem2))
            cps.append(pltpu.async_copy(
                t1_hbm.at[idx_v.at[gg]], lin_v.at[pl.ds(g * G, G)], sem1))
        for cp in cps:
            cp.wait()

        def row_body(r, resvec):
            base = r * F
            v = rows_v[base]
            acc = v
            sq = v * v
            for f in range(1, F):
                v = rows_v[base + f]
                acc = acc + v
                sq = sq + v * v
            l0 = lin_v[pl.ds(base, 16)]
            l1 = lin_v[pl.ds(base + 16, 16)]
            w = (acc * acc - sq) * 0.5 + l0 + l1 * tmask_v[...]
            cs = plsc.cumsum(w)
            m15 = lane_v[...] == jnp.full((16,), 15, jnp.int32)
            plsc.store_compressed(
                out_v.at[pl.ds(c * CHUNK + r, 16)], cs, mask=m15)
            return resvec

        lax.fori_loop(0, CHUNK, row_body, 0)
        return carry0

    lax.fori_loop(0, NCHUNK, chunk_body, 0)

    bv = bias_v[...]
    for j in range(ROWS_W // 16):
        z = out_v[pl.ds(j * 16, 16)] + bv
        out_v[pl.ds(j * 16, 16)] = 1.0 / (1.0 + jnp.exp(-z))
    pltpu.sync_copy(out_v.at[pl.ds(0, ROWS_W)],
                    out_hbm.at[pl.ds(wid * ROWS_W, ROWS_W)])


def kernel(x, table1, table2, bias):
    offsets = jnp.arange(F, dtype=x.dtype) * 100000
    idx = (x + offsets[None, :]).astype(jnp.int32).reshape(NW, IDX_W // G, G)
    t1 = table1.reshape(-1)
    bias16 = jnp.broadcast_to(bias.astype(jnp.float32), (16,))
    lane16 = jnp.asarray(np.arange(16), jnp.int32)
    tmask16 = jnp.asarray(np.arange(16) < (F - 16), jnp.float32)
    out = _fm_sc(idx, t1, table2, bias16, lane16, tmask16)
    return out[:, None]


# speedup vs baseline: 1.0025x; 1.0025x over previous
import functools

import jax
import jax.numpy as jnp
import numpy as np
from jax import lax
from jax.experimental import pallas as pl
from jax.experimental.pallas import tpu as pltpu
from jax.experimental.pallas import tpu_sc as plsc

B = 16384
F = 26
D = 16
NW = 32
ROWS_W = B // NW
CHUNK = 64
NCHUNK = ROWS_W // CHUNK
G = 128
NGC = CHUNK * F // G
IDX_W = ROWS_W * F

_mesh = plsc.VectorSubcoreMesh(core_axis_name="c", subcore_axis_name="s")


@functools.partial(
    pl.kernel,
    mesh=_mesh,
    out_type=jax.ShapeDtypeStruct((B,), jnp.float32),
    compiler_params=pltpu.CompilerParams(
        use_tc_tiling_on_sc=False, needs_layout_passes=False),
    scratch_types=[
        pltpu.VMEM((NCHUNK, CHUNK * F), jnp.int32),
        pltpu.VMEM((CHUNK * F, D), jnp.float32),
        pltpu.VMEM((CHUNK * F,), jnp.float32),
        pltpu.VMEM((ROWS_W + 16,), jnp.float32),
        pltpu.VMEM((16,), jnp.float32),
        pltpu.VMEM((16,), jnp.int32),
        pltpu.SemaphoreType.DMA,
        pltpu.SemaphoreType.DMA,
    ],
)
def _fm_sc(idx_hbm, t1_hbm, t2_hbm, bias_hbm, lane_hbm, out_hbm,
           idx_v, rows_v, lin_v, out_v, bias_v, lane_v, sem2, sem1):
    wid = lax.axis_index("s") * 2 + lax.axis_index("c")
    pltpu.sync_copy(idx_hbm.at[wid], idx_v)
    pltpu.sync_copy(bias_hbm, bias_v)
    pltpu.sync_copy(lane_hbm, lane_v)

    def chunk_body(c, carry0):
        idx_c = idx_v.at[c]
        cp2 = pltpu.async_copy(t2_hbm.at[idx_c], rows_v, sem2)
        cp1 = pltpu.async_copy(t1_hbm.at[idx_c], lin_v, sem1)
        cp2.wait()
        cp1.wait()

        def row_body(r, carry1):
            v = rows_v[r]
            acc = v
            sq = v * v
            for f in range(1, F):
                v = rows_v[f * CHUNK + r]
                acc = acc + v
                sq = sq + v * v
            w = (acc * acc - sq) * 0.5
            cs = plsc.cumsum(w)
            m15 = lane_v[...] == jnp.full((16,), 15, jnp.int32)
            plsc.store_compressed(
                out_v.at[pl.ds(c * CHUNK + r, 16)], cs, mask=m15)
            return carry1

        lax.fori_loop(0, CHUNK, row_body, 0)

        for q in range(CHUNK // 16):
            lin = lin_v[pl.ds(q * 16, 16)]
            for f in range(1, F):
                lin = lin + lin_v[pl.ds(f * CHUNK + q * 16, 16)]
            pos = c * CHUNK + q * 16
            out_v[pl.ds(pos, 16)] = out_v[pl.ds(pos, 16)] + lin
        return carry0

    lax.fori_loop(0, NCHUNK, chunk_body, 0)

    bv = bias_v[...]
    for j in range(ROWS_W // 16):
        z = out_v[pl.ds(j * 16, 16)] + bv
        out_v[pl.ds(j * 16, 16)] = 1.0 / (1.0 + jnp.exp(-z))
    pltpu.sync_copy(out_v.at[pl.ds(0, ROWS_W)],
                    out_hbm.at[pl.ds(wid * ROWS_W, ROWS_W)])


def kernel(x, table1, table2, bias):
    offsets = jnp.arange(F, dtype=x.dtype) * 100000
    idx = (x + offsets[None, :]).astype(jnp.int32)
    idx = idx.reshape(NW, NCHUNK, CHUNK, F).transpose(0, 1, 3, 2)
    idx = idx.reshape(NW, NCHUNK, CHUNK * F)
    t1 = table1.reshape(-1)
    bias16 = jnp.broadcast_to(bias.astype(jnp.float32), (16,))
    lane16 = jnp.asarray(np.arange(16), jnp.int32)
    out = _fm_sc(idx, t1, table2, bias16, lane16)
    return out[:, None]
